# baseline (speedup 1.0000x reference)
import functools

import jax
import jax.numpy as jnp
from jax import lax
from jax.experimental import pallas as pl
from jax.experimental.pallas import tpu as pltpu
from jax.experimental.pallas import tpu_sc as plsc

MAX_LEN = 200
EMB_DIM = 128
BATCH = 4096
NTOK = BATCH * MAX_LEN
HIDDEN = 32
PK = EMB_DIM // 2

_info = plsc.get_sparse_core_info()
_NC, _NS = _info.num_cores, _info.num_subcores
NW = _NC * _NS
ROWS_PER_W = NTOK // NW
CH = 128
NCHUNK = ROWS_PER_W // CH


def _make_sc_gather():
    mesh = plsc.VectorSubcoreMesh(core_axis_name="c", subcore_axis_name="s")

    @functools.partial(
        pl.kernel,
        mesh=mesh,
        out_type=jax.ShapeDtypeStruct((NTOK, PK), jnp.int32),
        scratch_types=[
            pltpu.VMEM((ROWS_PER_W,), jnp.int32),
            pltpu.VMEM((CH, PK), jnp.int32),
            pltpu.SemaphoreType.DMA,
        ],
        compiler_params=pltpu.CompilerParams(use_tc_tiling_on_sc=False),
    )
    def gather_k(idx_hbm, table_hbm, out_hbm, idx_v, rows_v, sem):
        wid = lax.axis_index("s") * _NC + lax.axis_index("c")
        base = wid * ROWS_PER_W
        pltpu.sync_copy(idx_hbm.at[pl.ds(base, ROWS_PER_W)], idx_v)

        def body(c, carry):
            off = c * CH
            pltpu.async_copy(
                table_hbm.at[idx_v.at[pl.ds(off, CH)]], rows_v, sem
            ).wait()
            pltpu.sync_copy(rows_v, out_hbm.at[pl.ds(base + off, CH)])
            return carry

        lax.fori_loop(0, NCHUNK, body, 0)

    return gather_k


_sc_gather = _make_sc_gather()

BB = 512
TT = 25
NT = MAX_LEN // TT


def _mlp_body(s_ref, w_ref, b1_ref, w2_ref, b2_ref, out_ref, acc_ref):
    tc = pl.program_id(1)
    z = pltpu.bitcast(s_ref[0], jnp.bfloat16)
    partial = jnp.dot(z, w_ref[0], preferred_element_type=jnp.float32)
    for tt in range(1, TT):
        z = pltpu.bitcast(s_ref[tt], jnp.bfloat16)
        partial += jnp.dot(z, w_ref[tt], preferred_element_type=jnp.float32)

    @pl.when(tc == 0)
    def _():
        acc_ref[...] = jnp.zeros_like(acc_ref)

    acc_ref[...] += partial

    @pl.when(tc == NT - 1)
    def _():
        r = lax.broadcasted_iota(jnp.int32, (BB, 2 * BB), 1)
        b = lax.broadcasted_iota(jnp.int32, (BB, 2 * BB), 0)
        s_even = (r == 2 * b).astype(jnp.float32)
        s_odd = (r == 2 * b + 1).astype(jnp.float32)
        acc = acc_ref[...]
        h = jnp.dot(s_even, acc[:, :HIDDEN], preferred_element_type=jnp.float32)
        h += jnp.dot(s_odd, acc[:, HIDDEN:], preferred_element_type=jnp.float32)
        h = jnp.maximum(h + b1_ref[...], 0.0)
        o = jnp.sum(h * w2_ref[...], axis=1, keepdims=True) + b2_ref[...]
        out_ref[...] = jax.nn.sigmoid(o)


def _mlp(staged3, W_all, b1r, W2r, b2r):
    return pl.pallas_call(
        _mlp_body,
        grid=(BATCH // BB, NT),
        in_specs=[
            pl.BlockSpec((TT, BB, PK), lambda i, t: (t, i, 0)),
            pl.BlockSpec((TT, PK, 2 * HIDDEN), lambda i, t: (t, 0, 0)),
            pl.BlockSpec((1, HIDDEN), lambda i, t: (0, 0)),
            pl.BlockSpec((1, HIDDEN), lambda i, t: (0, 0)),
            pl.BlockSpec((1, 1), lambda i, t: (0, 0)),
        ],
        out_specs=pl.BlockSpec((BB, 1), lambda i, t: (i, 0)),
        out_shape=jax.ShapeDtypeStruct((BATCH, 1), jnp.float32),
        scratch_shapes=[pltpu.VMEM((2 * BB, 2 * HIDDEN), jnp.float32)],
    )(staged3, W_all, b1r, W2r, b2r)


_LOW_TO_EVEN = True


def kernel(x, emb, W1, b1, W2, b2):
    idx = x.astype(jnp.int32).T.reshape(-1)
    table = lax.bitcast_convert_type(
        emb.astype(jnp.bfloat16).reshape(-1, PK, 2), jnp.int32
    )
    staged = _sc_gather(idx, table)
    staged3 = staged.reshape(MAX_LEN, BATCH, PK)
    W1r = W1.reshape(MAX_LEN, PK, 2, HIDDEN)
    if _LOW_TO_EVEN:
        W_all = jnp.concatenate([W1r[:, :, 0, :], W1r[:, :, 1, :]], axis=-1)
    else:
        W_all = jnp.concatenate([W1r[:, :, 1, :], W1r[:, :, 0, :]], axis=-1)
    W_all = W_all.astype(jnp.bfloat16)
    return _mlp(
        staged3,
        W_all,
        b1.reshape(1, HIDDEN),
        W2.reshape(1, HIDDEN),
        b2.reshape(1, 1),
    )


# speedup vs baseline: 2.5254x; 2.5254x over previous
import functools

import jax
import jax.numpy as jnp
from jax import lax
from jax.experimental import pallas as pl
from jax.experimental.pallas import tpu as pltpu
from jax.experimental.pallas import tpu_sc as plsc

MAX_LEN = 200
EMB_DIM = 128
BATCH = 4096
NTOK = BATCH * MAX_LEN
HIDDEN = 32

_info = plsc.get_sparse_core_info()
_NC, _NS = _info.num_cores, _info.num_subcores
NW = _NC * _NS
ROWS_PER_W = NTOK // NW
CH = 128
NCHUNK = ROWS_PER_W // CH


def _make_sc_gather():
    mesh = plsc.VectorSubcoreMesh(core_axis_name="c", subcore_axis_name="s")

    @functools.partial(
        pl.kernel,
        mesh=mesh,
        out_type=jax.ShapeDtypeStruct((NTOK, EMB_DIM), jnp.float32),
        scratch_types=[
            pltpu.VMEM((ROWS_PER_W,), jnp.int32),
            pltpu.VMEM((CH, EMB_DIM), jnp.float32),
            pltpu.VMEM((CH, EMB_DIM), jnp.float32),
            pltpu.SemaphoreType.DMA,
            pltpu.SemaphoreType.DMA,
            pltpu.SemaphoreType.DMA,
            pltpu.SemaphoreType.DMA,
        ],
    )
    def gather_k(idx_hbm, table_hbm, out_hbm, idx_v, rows0, rows1, g0, g1, o0, o1):
        wid = lax.axis_index("s") * _NC + lax.axis_index("c")
        base = wid * ROWS_PER_W
        pltpu.sync_copy(idx_hbm.at[pl.ds(base, ROWS_PER_W)], idx_v)

        def g_start(c, buf, sem):
            pltpu.async_copy(table_hbm.at[idx_v.at[pl.ds(c * CH, CH)]], buf, sem)

        def g_wait(buf, sem):
            pltpu.make_async_copy(
                table_hbm.at[idx_v.at[pl.ds(0, CH)]], buf, sem
            ).wait()

        def o_start(c, buf, sem):
            pltpu.async_copy(buf, out_hbm.at[pl.ds(base + c * CH, CH)], sem)

        def o_wait(buf, sem):
            pltpu.make_async_copy(
                buf, out_hbm.at[pl.ds(base, CH)], sem
            ).wait()

        g_start(0, rows0, g0)
        g_start(1, rows1, g1)

        def body(p, carry):
            c = 2 * p
            g_wait(rows0, g0)
            o_start(c, rows0, o0)
            g_wait(rows1, g1)
            o_start(c + 1, rows1, o1)

            @pl.when(p + 1 < NCHUNK // 2)
            def _():
                o_wait(rows0, o0)
                g_start(c + 2, rows0, g0)
                o_wait(rows1, o1)
                g_start(c + 3, rows1, g1)

            return carry

        lax.fori_loop(0, NCHUNK // 2, body, 0)
        o_wait(rows0, o0)
        o_wait(rows1, o1)

    return gather_k


_sc_gather = _make_sc_gather()

BB = 512
TT = 25
NT = MAX_LEN // TT


def _mlp_body(s_ref, w1_ref, b1_ref, w2_ref, b2_ref, out_ref, acc_ref):
    tc = pl.program_id(1)
    partial = jnp.dot(s_ref[0], w1_ref[0], preferred_element_type=jnp.float32)
    for tt in range(1, TT):
        partial += jnp.dot(
            s_ref[tt], w1_ref[tt], preferred_element_type=jnp.float32
        )

    @pl.when(tc == 0)
    def _():
        acc_ref[...] = jnp.zeros_like(acc_ref)

    acc_ref[...] += partial

    @pl.when(tc == NT - 1)
    def _():
        h = jnp.maximum(acc_ref[...] + b1_ref[...], 0.0)
        o = jnp.sum(h * w2_ref[...], axis=1, keepdims=True) + b2_ref[...]
        out_ref[...] = jax.nn.sigmoid(o)


def _mlp(staged3, W1r, b1r, W2r, b2r):
    return pl.pallas_call(
        _mlp_body,
        grid=(BATCH // BB, NT),
        in_specs=[
            pl.BlockSpec((TT, BB, EMB_DIM), lambda i, t: (t, i, 0)),
            pl.BlockSpec((TT, EMB_DIM, HIDDEN), lambda i, t: (t, 0, 0)),
            pl.BlockSpec((1, HIDDEN), lambda i, t: (0, 0)),
            pl.BlockSpec((1, HIDDEN), lambda i, t: (0, 0)),
            pl.BlockSpec((1, 1), lambda i, t: (0, 0)),
        ],
        out_specs=pl.BlockSpec((BB, 1), lambda i, t: (i, 0)),
        out_shape=jax.ShapeDtypeStruct((BATCH, 1), jnp.float32),
        scratch_shapes=[pltpu.VMEM((BB, HIDDEN), jnp.float32)],
    )(staged3, W1r, b1r, W2r, b2r)


def kernel(x, emb, W1, b1, W2, b2):
    idx = x.astype(jnp.int32).T.reshape(-1)
    staged = _sc_gather(idx, emb)
    staged3 = staged.reshape(MAX_LEN, BATCH, EMB_DIM)
    W1r = W1.reshape(MAX_LEN, EMB_DIM, HIDDEN)
    return _mlp(
        staged3,
        W1r,
        b1.reshape(1, HIDDEN),
        W2.reshape(1, HIDDEN),
        b2.reshape(1, 1),
    )


# speedup vs baseline: 2.5979x; 1.0287x over previous
import functools

import jax
import jax.numpy as jnp
from jax import lax
from jax.experimental import pallas as pl
from jax.experimental.pallas import tpu as pltpu
from jax.experimental.pallas import tpu_sc as plsc

MAX_LEN = 200
EMB_DIM = 128
BATCH = 4096
HIDDEN = 32
K = 2
BK = BATCH // K
NTOK_K = BK * MAX_LEN

_info = plsc.get_sparse_core_info()
_NC, _NS = _info.num_cores, _info.num_subcores
NW = _NC * _NS
ROWS_PER_W = NTOK_K // NW
CH = 128
NCHUNK = ROWS_PER_W // CH


def _make_sc_gather():
    mesh = plsc.VectorSubcoreMesh(core_axis_name="c", subcore_axis_name="s")

    @functools.partial(
        pl.kernel,
        mesh=mesh,
        out_type=jax.ShapeDtypeStruct((NTOK_K, EMB_DIM), jnp.float32),
        scratch_types=[
            pltpu.VMEM((ROWS_PER_W,), jnp.int32),
            pltpu.VMEM((CH, EMB_DIM), jnp.float32),
            pltpu.VMEM((CH, EMB_DIM), jnp.float32),
            pltpu.SemaphoreType.DMA,
            pltpu.SemaphoreType.DMA,
            pltpu.SemaphoreType.DMA,
            pltpu.SemaphoreType.DMA,
        ],
    )
    def gather_k(idx_hbm, table_hbm, out_hbm, idx_v, rows0, rows1, g0, g1, o0, o1):
        wid = lax.axis_index("s") * _NC + lax.axis_index("c")
        base = wid * ROWS_PER_W
        pltpu.sync_copy(idx_hbm.at[pl.ds(base, ROWS_PER_W)], idx_v)

        def g_start(c, buf, sem):
            pltpu.async_copy(table_hbm.at[idx_v.at[pl.ds(c * CH, CH)]], buf, sem)

        def g_wait(buf, sem):
            pltpu.make_async_copy(
                table_hbm.at[idx_v.at[pl.ds(0, CH)]], buf, sem
            ).wait()

        def o_start(c, buf, sem):
            pltpu.async_copy(buf, out_hbm.at[pl.ds(base + c * CH, CH)], sem)

        def o_wait(buf, sem):
            pltpu.make_async_copy(
                buf, out_hbm.at[pl.ds(base, CH)], sem
            ).wait()

        g_start(0, rows0, g0)
        g_start(1, rows1, g1)

        def body(p, carry):
            c = 2 * p
            g_wait(rows0, g0)
            o_start(c, rows0, o0)
            g_wait(rows1, g1)
            o_start(c + 1, rows1, o1)

            @pl.when(p + 1 < NCHUNK // 2)
            def _():
                o_wait(rows0, o0)
                g_start(c + 2, rows0, g0)
                o_wait(rows1, o1)
                g_start(c + 3, rows1, g1)

            return carry

        lax.fori_loop(0, NCHUNK // 2, body, 0)
        o_wait(rows0, o0)
        o_wait(rows1, o1)

    return gather_k


_sc_gather = _make_sc_gather()

BB = 512
TT = 25
NT = MAX_LEN // TT


def _mlp_body(s_ref, w1_ref, b1_ref, w2_ref, b2_ref, out_ref, acc_ref):
    tc = pl.program_id(1)
    partial = jnp.dot(s_ref[0], w1_ref[0], preferred_element_type=jnp.float32)
    for tt in range(1, TT):
        partial += jnp.dot(
            s_ref[tt], w1_ref[tt], preferred_element_type=jnp.float32
        )

    @pl.when(tc == 0)
    def _():
        acc_ref[...] = jnp.zeros_like(acc_ref)

    acc_ref[...] += partial

    @pl.when(tc == NT - 1)
    def _():
        h = jnp.maximum(acc_ref[...] + b1_ref[...], 0.0)
        o = jnp.sum(h * w2_ref[...], axis=1, keepdims=True) + b2_ref[...]
        out_ref[...] = jax.nn.sigmoid(o)


def _mlp(staged3, W1r, b1r, W2r, b2r):
    return pl.pallas_call(
        _mlp_body,
        grid=(BK // BB, NT),
        in_specs=[
            pl.BlockSpec((TT, BB, EMB_DIM), lambda i, t: (t, i, 0)),
            pl.BlockSpec((TT, EMB_DIM, HIDDEN), lambda i, t: (t, 0, 0)),
            pl.BlockSpec((1, HIDDEN), lambda i, t: (0, 0)),
            pl.BlockSpec((1, HIDDEN), lambda i, t: (0, 0)),
            pl.BlockSpec((1, 1), lambda i, t: (0, 0)),
        ],
        out_specs=pl.BlockSpec((BB, 1), lambda i, t: (i, 0)),
        out_shape=jax.ShapeDtypeStruct((BK, 1), jnp.float32),
        scratch_shapes=[pltpu.VMEM((BB, HIDDEN), jnp.float32)],
    )(staged3, W1r, b1r, W2r, b2r)


def kernel(x, emb, W1, b1, W2, b2):
    xT = x.astype(jnp.int32).T
    W1r = W1.reshape(MAX_LEN, EMB_DIM, HIDDEN)
    b1r = b1.reshape(1, HIDDEN)
    W2r = W2.reshape(1, HIDDEN)
    b2r = b2.reshape(1, 1)
    outs = []
    for k in range(K):
        idx_k = xT[:, k * BK:(k + 1) * BK].reshape(-1)
        staged = _sc_gather(idx_k, emb)
        staged3 = staged.reshape(MAX_LEN, BK, EMB_DIM)
        outs.append(_mlp(staged3, W1r, b1r, W2r, b2r))
    return jnp.concatenate(outs, axis=0)


# speedup vs baseline: 2.6834x; 1.0329x over previous
import functools

import jax
import jax.numpy as jnp
from jax import lax
from jax.experimental import pallas as pl
from jax.experimental.pallas import tpu as pltpu
from jax.experimental.pallas import tpu_sc as plsc

MAX_LEN = 200
EMB_DIM = 128
BATCH = 4096
HIDDEN = 32
PK = EMB_DIM // 2
K = 2
BK = BATCH // K
NTOK_K = BK * MAX_LEN
NQ = NTOK_K // 2

_info = plsc.get_sparse_core_info()
_NC, _NS = _info.num_cores, _info.num_subcores
NW = _NC * _NS
HW = NW // 2
ROWS_PER_W = NTOK_K // NW
CH = 128
NCHUNK = ROWS_PER_W // CH


def _make_sc_gather():
    mesh = plsc.VectorSubcoreMesh(core_axis_name="c", subcore_axis_name="s")

    @functools.partial(
        pl.kernel,
        mesh=mesh,
        out_type=jax.ShapeDtypeStruct((NQ, EMB_DIM), jnp.int32),
        scratch_types=[
            pltpu.VMEM((ROWS_PER_W,), jnp.int32),
            pltpu.VMEM((CH, PK), jnp.int32),
            pltpu.VMEM((CH, PK), jnp.int32),
            pltpu.SemaphoreType.DMA,
            pltpu.SemaphoreType.DMA,
            pltpu.SemaphoreType.DMA,
            pltpu.SemaphoreType.DMA,
        ],
        compiler_params=pltpu.CompilerParams(use_tc_tiling_on_sc=False),
    )
    def gather_k(idx_hbm, table_hbm, out_hbm, idx_v, rows0, rows1, g0, g1, o0, o1):
        wid = lax.axis_index("s") * _NC + lax.axis_index("c")
        base = wid * ROWS_PER_W
        qbase = (wid % HW) * ROWS_PER_W
        lane0 = (wid // HW) * PK
        pltpu.sync_copy(idx_hbm.at[pl.ds(base, ROWS_PER_W)], idx_v)

        def g_start(c, buf, sem):
            pltpu.async_copy(table_hbm.at[idx_v.at[pl.ds(c * CH, CH)]], buf, sem)

        def g_wait(buf, sem):
            pltpu.make_async_copy(
                table_hbm.at[idx_v.at[pl.ds(0, CH)]], buf, sem
            ).wait()

        def o_start(c, buf, sem):
            pltpu.async_copy(
                buf,
                out_hbm.at[pl.ds(qbase + c * CH, CH), pl.ds(lane0, PK)],
                sem,
            )

        def o_wait(buf, sem):
            pltpu.make_async_copy(
                buf, out_hbm.at[pl.ds(qbase, CH), pl.ds(lane0, PK)], sem
            ).wait()

        g_start(0, rows0, g0)
        g_start(1, rows1, g1)

        def body(p, carry):
            c = 2 * p
            g_wait(rows0, g0)
            o_start(c, rows0, o0)
            g_wait(rows1, g1)
            o_start(c + 1, rows1, o1)

            @pl.when(p + 1 < NCHUNK // 2)
            def _():
                o_wait(rows0, o0)
                g_start(c + 2, rows0, g0)
                o_wait(rows1, o1)
                g_start(c + 3, rows1, g1)

            return carry

        lax.fori_loop(0, NCHUNK // 2, body, 0)
        o_wait(rows0, o0)
        o_wait(rows1, o1)

    return gather_k


_sc_gather = _make_sc_gather()

BB = 512
BBH = BB // 2
TT = 25
NT = MAX_LEN // TT


def _mlp_body(s_ref, w_ref, b1_ref, w2_ref, b2_ref, out_ref, acc0_ref, acc1_ref):
    tc = pl.program_id(1)
    p0 = None
    p1 = None
    for tt in range(TT):
        z = pltpu.bitcast(s_ref[tt], jnp.bfloat16)
        z0 = z[:, :PK]
        z1 = z[:, PK:]
        d0 = jnp.dot(z0, w_ref[tt], preferred_element_type=jnp.float32)
        d1 = jnp.dot(z1, w_ref[tt], preferred_element_type=jnp.float32)
        p0 = d0 if p0 is None else p0 + d0
        p1 = d1 if p1 is None else p1 + d1

    @pl.when(tc == 0)
    def _():
        acc0_ref[...] = jnp.zeros_like(acc0_ref)
        acc1_ref[...] = jnp.zeros_like(acc1_ref)

    acc0_ref[...] += p0
    acc1_ref[...] += p1

    @pl.when(tc == NT - 1)
    def _():
        r = lax.broadcasted_iota(jnp.int32, (BB, BB), 1)
        b = lax.broadcasted_iota(jnp.int32, (BB, BB), 0)
        even = (b % 2) == 0
        se0 = (even & (r == b)).astype(jnp.float32)
        se1 = (even & (r == b + 1)).astype(jnp.float32)
        so0 = (~even & (r == b - 1)).astype(jnp.float32)
        so1 = (~even & (r == b)).astype(jnp.float32)
        a0 = acc0_ref[...]
        a1 = acc1_ref[...]
        h = jnp.dot(se0, a0[:, :HIDDEN], preferred_element_type=jnp.float32)
        h += jnp.dot(se1, a0[:, HIDDEN:], preferred_element_type=jnp.float32)
        h += jnp.dot(so0, a1[:, :HIDDEN], preferred_element_type=jnp.float32)
        h += jnp.dot(so1, a1[:, HIDDEN:], preferred_element_type=jnp.float32)
        h = jnp.maximum(h + b1_ref[...], 0.0)
        o = jnp.sum(h * w2_ref[...], axis=1, keepdims=True) + b2_ref[...]
        out_ref[...] = jax.nn.sigmoid(o)


def _mlp(staged3, W_all, b1r, W2r, b2r):
    return pl.pallas_call(
        _mlp_body,
        grid=(BK // BB, NT),
        in_specs=[
            pl.BlockSpec((TT, BBH, EMB_DIM), lambda i, t: (t, i, 0)),
            pl.BlockSpec((TT, PK, 2 * HIDDEN), lambda i, t: (t, 0, 0)),
            pl.BlockSpec((1, HIDDEN), lambda i, t: (0, 0)),
            pl.BlockSpec((1, HIDDEN), lambda i, t: (0, 0)),
            pl.BlockSpec((1, 1), lambda i, t: (0, 0)),
        ],
        out_specs=pl.BlockSpec((BB, 1), lambda i, t: (i, 0)),
        out_shape=jax.ShapeDtypeStruct((BK, 1), jnp.float32),
        scratch_shapes=[
            pltpu.VMEM((BB, 2 * HIDDEN), jnp.float32),
            pltpu.VMEM((BB, 2 * HIDDEN), jnp.float32),
        ],
    )(staged3, W_all, b1r, W2r, b2r)


def kernel(x, emb, W1, b1, W2, b2):
    xT = x.astype(jnp.int32).T
    ebf = emb.astype(jnp.bfloat16)
    lo = lax.bitcast_convert_type(ebf[:, :PK], jnp.uint16).astype(jnp.uint32)
    hi = lax.bitcast_convert_type(ebf[:, PK:], jnp.uint16).astype(jnp.uint32)
    table = lax.bitcast_convert_type(lo | (hi << 16), jnp.int32)
    W1r = W1.reshape(MAX_LEN, 2, PK, HIDDEN)
    W_all = jnp.concatenate([W1r[:, 0], W1r[:, 1]], axis=-1).astype(jnp.bfloat16)
    b1r = b1.reshape(1, HIDDEN)
    W2r = W2.reshape(1, HIDDEN)
    b2r = b2.reshape(1, 1)
    outs = []
    for k in range(K):
        xk = xT[:, k * BK:(k + 1) * BK]
        idx_k = jnp.concatenate(
            [xk[:, 0::2].reshape(-1), xk[:, 1::2].reshape(-1)]
        )
        staged = _sc_gather(idx_k, table)
        staged3 = staged.reshape(MAX_LEN, BK // 2, EMB_DIM)
        outs.append(_mlp(staged3, W_all, b1r, W2r, b2r))
    return jnp.concatenate(outs, axis=0)


# speedup vs baseline: 3.1078x; 1.1582x over previous
import functools

import jax
import jax.numpy as jnp
from jax import lax
from jax.experimental import pallas as pl
from jax.experimental.pallas import tpu as pltpu
from jax.experimental.pallas import tpu_sc as plsc

MAX_LEN = 200
EMB_DIM = 128
BATCH = 4096
HIDDEN = 32
PK = EMB_DIM // 2
K = 2
BK = BATCH // K
NTOK_K = BK * MAX_LEN
NQ = NTOK_K // 2

_info = plsc.get_sparse_core_info()
_NC, _NS = _info.num_cores, _info.num_subcores
NW = _NC * _NS
HW = NW // 2
ROWS_PER_W = NTOK_K // NW
CH = 128
NCHUNK = ROWS_PER_W // CH


def _make_sc_gather():
    mesh = plsc.VectorSubcoreMesh(core_axis_name="c", subcore_axis_name="s")

    @functools.partial(
        pl.kernel,
        mesh=mesh,
        out_type=jax.ShapeDtypeStruct((NQ, EMB_DIM), jnp.int32),
        scratch_types=[
            pltpu.VMEM((ROWS_PER_W,), jnp.int32),
            pltpu.VMEM((CH, PK), jnp.int32),
            pltpu.VMEM((CH, PK), jnp.int32),
            pltpu.SemaphoreType.DMA,
            pltpu.SemaphoreType.DMA,
            pltpu.SemaphoreType.DMA,
            pltpu.SemaphoreType.DMA,
        ],
        compiler_params=pltpu.CompilerParams(use_tc_tiling_on_sc=False),
    )
    def gather_k(idx_hbm, table_hbm, out_hbm, idx_v, rows0, rows1, g0, g1, o0, o1):
        wid = lax.axis_index("s") * _NC + lax.axis_index("c")
        base = wid * ROWS_PER_W
        qbase = (wid % HW) * ROWS_PER_W
        lane0 = (wid // HW) * PK
        pltpu.sync_copy(idx_hbm.at[pl.ds(base, ROWS_PER_W)], idx_v)

        def g_start(c, buf, sem):
            pltpu.async_copy(table_hbm.at[idx_v.at[pl.ds(c * CH, CH)]], buf, sem)

        def g_wait(buf, sem):
            pltpu.make_async_copy(
                table_hbm.at[idx_v.at[pl.ds(0, CH)]], buf, sem
            ).wait()

        def o_start(c, buf, sem):
            pltpu.async_copy(
                buf,
                out_hbm.at[pl.ds(qbase + c * CH, CH), pl.ds(lane0, PK)],
                sem,
            )

        def o_wait(buf, sem):
            pltpu.make_async_copy(
                buf, out_hbm.at[pl.ds(qbase, CH), pl.ds(lane0, PK)], sem
            ).wait()

        g_start(0, rows0, g0)
        g_start(1, rows1, g1)

        def body(p, carry):
            c = 2 * p
            g_wait(rows0, g0)
            o_start(c, rows0, o0)
            g_wait(rows1, g1)
            o_start(c + 1, rows1, o1)

            @pl.when(p + 1 < NCHUNK // 2)
            def _():
                o_wait(rows0, o0)
                g_start(c + 2, rows0, g0)
                o_wait(rows1, o1)
                g_start(c + 3, rows1, g1)

            return carry

        lax.fori_loop(0, NCHUNK // 2, body, 0)
        o_wait(rows0, o0)
        o_wait(rows1, o1)

    return gather_k


_sc_gather = _make_sc_gather()

VOCAB_BLK = 2000


def _pack_body(e_ref, t_ref):
    eb = e_ref[...].astype(jnp.bfloat16)
    lo = lax.bitcast_convert_type(eb[:, :PK], jnp.uint16).astype(jnp.uint32)
    hi = lax.bitcast_convert_type(eb[:, PK:], jnp.uint16).astype(jnp.uint32)
    t_ref[...] = lax.bitcast_convert_type(lo | (hi << 16), jnp.int32)


def _pack(emb):
    v = emb.shape[0]
    return pl.pallas_call(
        _pack_body,
        grid=(v // VOCAB_BLK,),
        in_specs=[pl.BlockSpec((VOCAB_BLK, EMB_DIM), lambda i: (i, 0))],
        out_specs=pl.BlockSpec((VOCAB_BLK, PK), lambda i: (i, 0)),
        out_shape=jax.ShapeDtypeStruct((v, PK), jnp.int32),
    )(emb)

BB = 512
BBH = BB // 2
TT = 25
NT = MAX_LEN // TT


def _mlp_body(s_ref, w_ref, b1_ref, w2_ref, b2_ref, out_ref, acc0_ref, acc1_ref):
    tc = pl.program_id(1)
    p0 = None
    p1 = None
    for tt in range(TT):
        z = pltpu.bitcast(s_ref[tt], jnp.bfloat16)
        z0 = z[:, :PK]
        z1 = z[:, PK:]
        d0 = jnp.dot(z0, w_ref[tt], preferred_element_type=jnp.float32)
        d1 = jnp.dot(z1, w_ref[tt], preferred_element_type=jnp.float32)
        p0 = d0 if p0 is None else p0 + d0
        p1 = d1 if p1 is None else p1 + d1

    @pl.when(tc == 0)
    def _():
        acc0_ref[...] = jnp.zeros_like(acc0_ref)
        acc1_ref[...] = jnp.zeros_like(acc1_ref)

    acc0_ref[...] += p0
    acc1_ref[...] += p1

    @pl.when(tc == NT - 1)
    def _():
        r = lax.broadcasted_iota(jnp.int32, (BB, BB), 1)
        b = lax.broadcasted_iota(jnp.int32, (BB, BB), 0)
        even = (b % 2) == 0
        se0 = (even & (r == b)).astype(jnp.float32)
        se1 = (even & (r == b + 1)).astype(jnp.float32)
        so0 = (~even & (r == b - 1)).astype(jnp.float32)
        so1 = (~even & (r == b)).astype(jnp.float32)
        a0 = acc0_ref[...]
        a1 = acc1_ref[...]
        h = jnp.dot(se0, a0[:, :HIDDEN], preferred_element_type=jnp.float32)
        h += jnp.dot(se1, a0[:, HIDDEN:], preferred_element_type=jnp.float32)
        h += jnp.dot(so0, a1[:, :HIDDEN], preferred_element_type=jnp.float32)
        h += jnp.dot(so1, a1[:, HIDDEN:], preferred_element_type=jnp.float32)
        h = jnp.maximum(h + b1_ref[...], 0.0)
        o = jnp.sum(h * w2_ref[...], axis=1, keepdims=True) + b2_ref[...]
        out_ref[...] = jax.nn.sigmoid(o)


def _mlp(staged3, W_all, b1r, W2r, b2r):
    return pl.pallas_call(
        _mlp_body,
        grid=(BK // BB, NT),
        in_specs=[
            pl.BlockSpec((TT, BBH, EMB_DIM), lambda i, t: (t, i, 0)),
            pl.BlockSpec((TT, PK, 2 * HIDDEN), lambda i, t: (t, 0, 0)),
            pl.BlockSpec((1, HIDDEN), lambda i, t: (0, 0)),
            pl.BlockSpec((1, HIDDEN), lambda i, t: (0, 0)),
            pl.BlockSpec((1, 1), lambda i, t: (0, 0)),
        ],
        out_specs=pl.BlockSpec((BB, 1), lambda i, t: (i, 0)),
        out_shape=jax.ShapeDtypeStruct((BK, 1), jnp.float32),
        scratch_shapes=[
            pltpu.VMEM((BB, 2 * HIDDEN), jnp.float32),
            pltpu.VMEM((BB, 2 * HIDDEN), jnp.float32),
        ],
    )(staged3, W_all, b1r, W2r, b2r)


def kernel(x, emb, W1, b1, W2, b2):
    xT = x.astype(jnp.int32).T
    table = _pack(emb)
    W1r = W1.reshape(MAX_LEN, 2, PK, HIDDEN)
    W_all = jnp.concatenate([W1r[:, 0], W1r[:, 1]], axis=-1).astype(jnp.bfloat16)
    b1r = b1.reshape(1, HIDDEN)
    W2r = W2.reshape(1, HIDDEN)
    b2r = b2.reshape(1, 1)
    outs = []
    bk2 = BK // 2
    for k in range(K):
        xk = xT[:, k * BK:(k + 1) * BK]
        idx_k = jnp.concatenate(
            [xk[:, :bk2].reshape(-1), xk[:, bk2:].reshape(-1)]
        )
        staged = _sc_gather(idx_k, table)
        staged3 = staged.reshape(MAX_LEN, bk2, EMB_DIM)
        op = _mlp(staged3, W_all, b1r, W2r, b2r)
        outs.append(op.reshape(bk2, 2).T.reshape(BK, 1))
    return jnp.concatenate(outs, axis=0)


# speedup vs baseline: 3.3402x; 1.0748x over previous
import functools

import jax
import jax.numpy as jnp
from jax import lax
from jax.experimental import pallas as pl
from jax.experimental.pallas import tpu as pltpu
from jax.experimental.pallas import tpu_sc as plsc

MAX_LEN = 200
EMB_DIM = 128
BATCH = 4096
HIDDEN = 32
PK = EMB_DIM // 2
K = 4
BK = BATCH // K
NTOK_K = BK * MAX_LEN
NQ = NTOK_K // 2

_info = plsc.get_sparse_core_info()
_NC, _NS = _info.num_cores, _info.num_subcores
NW = _NC * _NS
HW = NW // 2
ROWS_PER_W = NTOK_K // NW
CH = 128
NCHUNK = ROWS_PER_W // CH


def _make_sc_gather():
    mesh = plsc.VectorSubcoreMesh(core_axis_name="c", subcore_axis_name="s")

    @functools.partial(
        pl.kernel,
        mesh=mesh,
        out_type=jax.ShapeDtypeStruct((NQ, EMB_DIM), jnp.int32),
        scratch_types=[
            pltpu.VMEM((ROWS_PER_W,), jnp.int32),
            pltpu.VMEM((CH, PK), jnp.int32),
            pltpu.VMEM((CH, PK), jnp.int32),
            pltpu.SemaphoreType.DMA,
            pltpu.SemaphoreType.DMA,
            pltpu.SemaphoreType.DMA,
            pltpu.SemaphoreType.DMA,
        ],
        compiler_params=pltpu.CompilerParams(use_tc_tiling_on_sc=False),
    )
    def gather_k(idx_hbm, table_hbm, out_hbm, idx_v, rows0, rows1, g0, g1, o0, o1):
        wid = lax.axis_index("s") * _NC + lax.axis_index("c")
        base = wid * ROWS_PER_W
        qbase = (wid % HW) * ROWS_PER_W
        lane0 = (wid // HW) * PK
        pltpu.sync_copy(idx_hbm.at[pl.ds(base, ROWS_PER_W)], idx_v)

        def g_start(c, buf, sem):
            pltpu.async_copy(table_hbm.at[idx_v.at[pl.ds(c * CH, CH)]], buf, sem)

        def g_wait(buf, sem):
            pltpu.make_async_copy(
                table_hbm.at[idx_v.at[pl.ds(0, CH)]], buf, sem
            ).wait()

        def o_start(c, buf, sem):
            pltpu.async_copy(
                buf,
                out_hbm.at[pl.ds(qbase + c * CH, CH), pl.ds(lane0, PK)],
                sem,
            )

        def o_wait(buf, sem):
            pltpu.make_async_copy(
                buf, out_hbm.at[pl.ds(qbase, CH), pl.ds(lane0, PK)], sem
            ).wait()

        g_start(0, rows0, g0)
        g_start(1, rows1, g1)

        def body(p, carry):
            c = 2 * p
            g_wait(rows0, g0)
            o_start(c, rows0, o0)
            g_wait(rows1, g1)
            o_start(c + 1, rows1, o1)

            @pl.when(p + 1 < NCHUNK // 2)
            def _():
                o_wait(rows0, o0)
                g_start(c + 2, rows0, g0)
                o_wait(rows1, o1)
                g_start(c + 3, rows1, g1)

            return carry

        lax.fori_loop(0, NCHUNK // 2, body, 0)
        o_wait(rows0, o0)
        o_wait(rows1, o1)

    return gather_k


_sc_gather = _make_sc_gather()

VOCAB_BLK = 4000


def _pack_body(e_ref, t_ref):
    eb = e_ref[...].astype(jnp.bfloat16)
    lo = lax.bitcast_convert_type(eb[:, :PK], jnp.uint16).astype(jnp.uint32)
    hi = lax.bitcast_convert_type(eb[:, PK:], jnp.uint16).astype(jnp.uint32)
    t_ref[...] = lax.bitcast_convert_type(lo | (hi << 16), jnp.int32)


def _pack(emb):
    v = emb.shape[0]
    return pl.pallas_call(
        _pack_body,
        grid=(v // VOCAB_BLK,),
        in_specs=[pl.BlockSpec((VOCAB_BLK, EMB_DIM), lambda i: (i, 0))],
        out_specs=pl.BlockSpec((VOCAB_BLK, PK), lambda i: (i, 0)),
        out_shape=jax.ShapeDtypeStruct((v, PK), jnp.int32),
    )(emb)

BB = 512
BBH = BB // 2
TT = 25
NT = MAX_LEN // TT


def _mlp_body(s_ref, w_ref, b1_ref, w2_ref, b2_ref, out_ref, acc0_ref, acc1_ref):
    tc = pl.program_id(1)
    p0 = None
    p1 = None
    for tt in range(TT):
        z = pltpu.bitcast(s_ref[tt], jnp.bfloat16)
        z0 = z[:, :PK]
        z1 = z[:, PK:]
        d0 = jnp.dot(z0, w_ref[tt], preferred_element_type=jnp.float32)
        d1 = jnp.dot(z1, w_ref[tt], preferred_element_type=jnp.float32)
        p0 = d0 if p0 is None else p0 + d0
        p1 = d1 if p1 is None else p1 + d1

    @pl.when(tc == 0)
    def _():
        acc0_ref[...] = jnp.zeros_like(acc0_ref)
        acc1_ref[...] = jnp.zeros_like(acc1_ref)

    acc0_ref[...] += p0
    acc1_ref[...] += p1

    @pl.when(tc == NT - 1)
    def _():
        r = lax.broadcasted_iota(jnp.int32, (BB, BB), 1)
        b = lax.broadcasted_iota(jnp.int32, (BB, BB), 0)
        even = (b % 2) == 0
        se0 = (even & (r == b)).astype(jnp.float32)
        se1 = (even & (r == b + 1)).astype(jnp.float32)
        so0 = (~even & (r == b - 1)).astype(jnp.float32)
        so1 = (~even & (r == b)).astype(jnp.float32)
        a0 = acc0_ref[...]
        a1 = acc1_ref[...]
        h = jnp.dot(se0, a0[:, :HIDDEN], preferred_element_type=jnp.float32)
        h += jnp.dot(se1, a0[:, HIDDEN:], preferred_element_type=jnp.float32)
        h += jnp.dot(so0, a1[:, :HIDDEN], preferred_element_type=jnp.float32)
        h += jnp.dot(so1, a1[:, HIDDEN:], preferred_element_type=jnp.float32)
        h = jnp.maximum(h + b1_ref[...], 0.0)
        o = jnp.sum(h * w2_ref[...], axis=1, keepdims=True) + b2_ref[...]
        out_ref[...] = jax.nn.sigmoid(o)


def _mlp(staged3, W_all, b1r, W2r, b2r):
    return pl.pallas_call(
        _mlp_body,
        grid=(BK // BB, NT),
        in_specs=[
            pl.BlockSpec((TT, BBH, EMB_DIM), lambda i, t: (t, i, 0)),
            pl.BlockSpec((TT, PK, 2 * HIDDEN), lambda i, t: (t, 0, 0)),
            pl.BlockSpec((1, HIDDEN), lambda i, t: (0, 0)),
            pl.BlockSpec((1, HIDDEN), lambda i, t: (0, 0)),
            pl.BlockSpec((1, 1), lambda i, t: (0, 0)),
        ],
        out_specs=pl.BlockSpec((BB, 1), lambda i, t: (i, 0)),
        out_shape=jax.ShapeDtypeStruct((BK, 1), jnp.float32),
        scratch_shapes=[
            pltpu.VMEM((BB, 2 * HIDDEN), jnp.float32),
            pltpu.VMEM((BB, 2 * HIDDEN), jnp.float32),
        ],
    )(staged3, W_all, b1r, W2r, b2r)


def kernel(x, emb, W1, b1, W2, b2):
    xT = x.astype(jnp.int32).T
    table = _pack(emb)
    W1r = W1.reshape(MAX_LEN, 2, PK, HIDDEN)
    W_all = jnp.concatenate([W1r[:, 0], W1r[:, 1]], axis=-1).astype(jnp.bfloat16)
    b1r = b1.reshape(1, HIDDEN)
    W2r = W2.reshape(1, HIDDEN)
    b2r = b2.reshape(1, 1)
    outs = []
    bk2 = BK // 2
    for k in range(K):
        xk = xT[:, k * BK:(k + 1) * BK]
        idx_k = jnp.concatenate(
            [xk[:, :bk2].reshape(-1), xk[:, bk2:].reshape(-1)]
        )
        staged = _sc_gather(idx_k, table)
        staged3 = staged.reshape(MAX_LEN, bk2, EMB_DIM)
        op = _mlp(staged3, W_all, b1r, W2r, b2r)
        outs.append(op.reshape(bk2, 2).T.reshape(BK, 1))
    return jnp.concatenate(outs, axis=0)


# speedup vs baseline: 3.3816x; 1.0124x over previous
import functools

import jax
import jax.numpy as jnp
from jax import lax
from jax.experimental import pallas as pl
from jax.experimental.pallas import tpu as pltpu
from jax.experimental.pallas import tpu_sc as plsc

MAX_LEN = 200
EMB_DIM = 128
BATCH = 4096
HIDDEN = 32
PK = EMB_DIM // 2
K = 8
BK = BATCH // K
NTOK_K = BK * MAX_LEN
NQ = NTOK_K // 2

_info = plsc.get_sparse_core_info()
_NC, _NS = _info.num_cores, _info.num_subcores
NW = _NC * _NS
HW = NW // 2
ROWS_PER_W = NTOK_K // NW
CH = 128
NCHUNK = ROWS_PER_W // CH


def _make_sc_gather():
    mesh = plsc.VectorSubcoreMesh(core_axis_name="c", subcore_axis_name="s")

    @functools.partial(
        pl.kernel,
        mesh=mesh,
        out_type=jax.ShapeDtypeStruct((NQ, EMB_DIM), jnp.int32),
        scratch_types=[
            pltpu.VMEM((ROWS_PER_W,), jnp.int32),
            pltpu.VMEM((CH, PK), jnp.int32),
            pltpu.VMEM((CH, PK), jnp.int32),
            pltpu.SemaphoreType.DMA,
            pltpu.SemaphoreType.DMA,
            pltpu.SemaphoreType.DMA,
            pltpu.SemaphoreType.DMA,
        ],
        compiler_params=pltpu.CompilerParams(use_tc_tiling_on_sc=False),
    )
    def gather_k(idx_hbm, table_hbm, out_hbm, idx_v, rows0, rows1, g0, g1, o0, o1):
        wid = lax.axis_index("s") * _NC + lax.axis_index("c")
        base = wid * ROWS_PER_W
        qbase = (wid % HW) * ROWS_PER_W
        lane0 = (wid // HW) * PK
        pltpu.sync_copy(idx_hbm.at[pl.ds(base, ROWS_PER_W)], idx_v)

        def g_start(c, buf, sem):
            pltpu.async_copy(table_hbm.at[idx_v.at[pl.ds(c * CH, CH)]], buf, sem)

        def g_wait(buf, sem):
            pltpu.make_async_copy(
                table_hbm.at[idx_v.at[pl.ds(0, CH)]], buf, sem
            ).wait()

        def o_start(c, buf, sem):
            pltpu.async_copy(
                buf,
                out_hbm.at[pl.ds(qbase + c * CH, CH), pl.ds(lane0, PK)],
                sem,
            )

        def o_wait(buf, sem):
            pltpu.make_async_copy(
                buf, out_hbm.at[pl.ds(qbase, CH), pl.ds(lane0, PK)], sem
            ).wait()

        g_start(0, rows0, g0)
        g_start(1, rows1, g1)

        def body(p, carry):
            c = 2 * p
            g_wait(rows0, g0)
            o_start(c, rows0, o0)
            g_wait(rows1, g1)
            o_start(c + 1, rows1, o1)

            @pl.when(p + 1 < NCHUNK // 2)
            def _():
                o_wait(rows0, o0)
                g_start(c + 2, rows0, g0)
                o_wait(rows1, o1)
                g_start(c + 3, rows1, g1)

            return carry

        lax.fori_loop(0, NCHUNK // 2, body, 0)
        o_wait(rows0, o0)
        o_wait(rows1, o1)

    return gather_k


_sc_gather = _make_sc_gather()

VOCAB_BLK = 4000


def _pack_body(e_ref, t_ref):
    eb = e_ref[...].astype(jnp.bfloat16)
    lo = lax.bitcast_convert_type(eb[:, :PK], jnp.uint16).astype(jnp.uint32)
    hi = lax.bitcast_convert_type(eb[:, PK:], jnp.uint16).astype(jnp.uint32)
    t_ref[...] = lax.bitcast_convert_type(lo | (hi << 16), jnp.int32)


def _pack(emb):
    v = emb.shape[0]
    return pl.pallas_call(
        _pack_body,
        grid=(v // VOCAB_BLK,),
        in_specs=[pl.BlockSpec((VOCAB_BLK, EMB_DIM), lambda i: (i, 0))],
        out_specs=pl.BlockSpec((VOCAB_BLK, PK), lambda i: (i, 0)),
        out_shape=jax.ShapeDtypeStruct((v, PK), jnp.int32),
    )(emb)

BB = 512
BBH = BB // 2
TT = 25
NT = MAX_LEN // TT


def _mlp_body(s_ref, w_ref, b1_ref, w2_ref, b2_ref, out_ref, acc0_ref, acc1_ref):
    tc = pl.program_id(1)
    p0 = None
    p1 = None
    for tt in range(TT):
        z = pltpu.bitcast(s_ref[tt], jnp.bfloat16)
        z0 = z[:, :PK]
        z1 = z[:, PK:]
        d0 = jnp.dot(z0, w_ref[tt], preferred_element_type=jnp.float32)
        d1 = jnp.dot(z1, w_ref[tt], preferred_element_type=jnp.float32)
        p0 = d0 if p0 is None else p0 + d0
        p1 = d1 if p1 is None else p1 + d1

    @pl.when(tc == 0)
    def _():
        acc0_ref[...] = jnp.zeros_like(acc0_ref)
        acc1_ref[...] = jnp.zeros_like(acc1_ref)

    acc0_ref[...] += p0
    acc1_ref[...] += p1

    @pl.when(tc == NT - 1)
    def _():
        r = lax.broadcasted_iota(jnp.int32, (BB, BB), 1)
        b = lax.broadcasted_iota(jnp.int32, (BB, BB), 0)
        even = (b % 2) == 0
        se0 = (even & (r == b)).astype(jnp.float32)
        se1 = (even & (r == b + 1)).astype(jnp.float32)
        so0 = (~even & (r == b - 1)).astype(jnp.float32)
        so1 = (~even & (r == b)).astype(jnp.float32)
        a0 = acc0_ref[...]
        a1 = acc1_ref[...]
        h = jnp.dot(se0, a0[:, :HIDDEN], preferred_element_type=jnp.float32)
        h += jnp.dot(se1, a0[:, HIDDEN:], preferred_element_type=jnp.float32)
        h += jnp.dot(so0, a1[:, :HIDDEN], preferred_element_type=jnp.float32)
        h += jnp.dot(so1, a1[:, HIDDEN:], preferred_element_type=jnp.float32)
        h = jnp.maximum(h + b1_ref[...], 0.0)
        o = jnp.sum(h * w2_ref[...], axis=1, keepdims=True) + b2_ref[...]
        out_ref[...] = jax.nn.sigmoid(o)


def _mlp(staged3, W_all, b1r, W2r, b2r):
    return pl.pallas_call(
        _mlp_body,
        grid=(BK // BB, NT),
        in_specs=[
            pl.BlockSpec((TT, BBH, EMB_DIM), lambda i, t: (t, i, 0)),
            pl.BlockSpec((TT, PK, 2 * HIDDEN), lambda i, t: (t, 0, 0)),
            pl.BlockSpec((1, HIDDEN), lambda i, t: (0, 0)),
            pl.BlockSpec((1, HIDDEN), lambda i, t: (0, 0)),
            pl.BlockSpec((1, 1), lambda i, t: (0, 0)),
        ],
        out_specs=pl.BlockSpec((BB, 1), lambda i, t: (i, 0)),
        out_shape=jax.ShapeDtypeStruct((BK, 1), jnp.float32),
        scratch_shapes=[
            pltpu.VMEM((BB, 2 * HIDDEN), jnp.float32),
            pltpu.VMEM((BB, 2 * HIDDEN), jnp.float32),
        ],
    )(staged3, W_all, b1r, W2r, b2r)


def kernel(x, emb, W1, b1, W2, b2):
    xT = x.astype(jnp.int32).T
    table = _pack(emb)
    W1r = W1.reshape(MAX_LEN, 2, PK, HIDDEN)
    W_all = jnp.concatenate([W1r[:, 0], W1r[:, 1]], axis=-1).astype(jnp.bfloat16)
    b1r = b1.reshape(1, HIDDEN)
    W2r = W2.reshape(1, HIDDEN)
    b2r = b2.reshape(1, 1)
    outs = []
    bk2 = BK // 2
    for k in range(K):
        xk = xT[:, k * BK:(k + 1) * BK]
        idx_k = jnp.concatenate(
            [xk[:, :bk2].reshape(-1), xk[:, bk2:].reshape(-1)]
        )
        staged = _sc_gather(idx_k, table)
        staged3 = staged.reshape(MAX_LEN, bk2, EMB_DIM)
        op = _mlp(staged3, W_all, b1r, W2r, b2r)
        outs.append(op.reshape(bk2, 2).T.reshape(BK, 1))
    return jnp.concatenate(outs, axis=0)
